# baseline (speedup 1.0000x reference)
import functools

import jax
import jax.numpy as jnp
from jax import lax
from jax.experimental import pallas as pl
from jax.experimental.pallas import tpu as pltpu
from jax.experimental.pallas import tpu_sc as plsc

NC, NS = 2, 16
NW = NC * NS
CHUNK = 128
FAST_SHARE = 0.39
EPS = 1e-5


def _pad_nodes(n):
    per_tile = -(-n // (NS * 8)) * 8 + 8
    return per_tile * NS


def _sc_mesh():
    return plsc.VectorSubcoreMesh(core_axis_name="c", subcore_axis_name="s",
                                  num_cores=NC, num_subcores=NS)


def _make_degree_kernel(n_pad, k0, k1):
    rows_per_tile = n_pad // NS
    pieces = rows_per_tile // CHUNK
    kmax = max(k0, k1)

    @functools.partial(
        pl.kernel,
        out_type=jax.ShapeDtypeStruct((NC, n_pad, CHUNK), jnp.float32),
        mesh=_sc_mesh(),
        scratch_types=[
            pltpu.VMEM_SHARED((n_pad, CHUNK), jnp.float32),
            pltpu.VMEM((kmax, CHUNK), jnp.int32),
            pltpu.VMEM((CHUNK, CHUNK), jnp.float32),
        ],
    )
    def deg_kernel(dst_hbm, ones_hbm, zeros_hbm, out_hbm,
                   deg_sh, dst_l, buf):
        c = lax.axis_index("c")
        s = lax.axis_index("s")
        base = s * rows_per_tile
        row0 = jnp.where(c == 0, s * k0, NS * k0 + s * k1)
        kc = jnp.where(c == 0, k0, k1)
        pltpu.sync_copy(zeros_hbm, buf)
        for p in range(pieces):
            pltpu.sync_copy(buf, deg_sh.at[pl.ds(base + p * CHUNK, CHUNK)])
        pltpu.sync_copy(ones_hbm, buf)
        pltpu.sync_copy(dst_hbm.at[pl.ds(row0, kmax)], dst_l)
        plsc.subcore_barrier()

        def step(j, carry):
            pltpu.sync_copy(buf, deg_sh.at[dst_l.at[j]], add=True)
            return carry

        lax.fori_loop(0, kc, step, 0)
        plsc.subcore_barrier()
        for p in range(pieces):
            pltpu.sync_copy(deg_sh.at[pl.ds(base + p * CHUNK, CHUNK)], buf)
            pltpu.sync_copy(
                buf, out_hbm.at[c].at[pl.ds(base + p * CHUNK, CHUNK)])

    return deg_kernel


def _make_scatter_kernel(n_pad, k0, k1, d):
    rows_per_tile = n_pad // NS
    pieces = rows_per_tile // CHUNK
    kmax = max(k0, k1)

    @functools.partial(
        pl.kernel,
        out_type=jax.ShapeDtypeStruct((NC, n_pad, d), jnp.float32),
        mesh=_sc_mesh(),
        scratch_types=[
            pltpu.VMEM_SHARED((n_pad, d), jnp.float32),
            pltpu.VMEM((kmax, CHUNK), jnp.int32),
            pltpu.VMEM((kmax, CHUNK), jnp.int32),
            pltpu.VMEM((CHUNK, d), jnp.float32),
            pltpu.SemaphoreType.DMA,
        ],
    )
    def mp_kernel(g_hbm, src_hbm, dst_hbm, out_hbm,
                  acc_sh, src_l, dst_l, buf, sem):
        c = lax.axis_index("c")
        s = lax.axis_index("s")
        base = s * rows_per_tile
        row0 = jnp.where(c == 0, s * k0, NS * k0 + s * k1)
        kc = jnp.where(c == 0, k0, k1)
        pltpu.sync_copy(src_hbm.at[pl.ds(row0, kmax)], src_l)
        pltpu.sync_copy(dst_hbm.at[pl.ds(row0, kmax)], dst_l)
        for p in range(pieces):
            pltpu.sync_copy(g_hbm.at[pl.ds(base + p * CHUNK, CHUNK)], buf)
            pltpu.sync_copy(buf, acc_sh.at[pl.ds(base + p * CHUNK, CHUNK)])
        plsc.subcore_barrier()

        def step(j, carry):
            pltpu.async_copy(g_hbm.at[src_l.at[j]], buf, sem).wait()
            pltpu.sync_copy(buf, acc_sh.at[dst_l.at[j]], add=True)
            return carry

        lax.fori_loop(0, kc, step, 0)
        plsc.subcore_barrier()
        for p in range(pieces):
            pltpu.sync_copy(acc_sh.at[pl.ds(base + p * CHUNK, CHUNK)], buf)
            pltpu.sync_copy(
                buf, out_hbm.at[c].at[pl.ds(base + p * CHUNK, CHUNK)])

    return mp_kernel


def _tc_transform(x_pad, W, c0, c1):
    def body(x_ref, w_ref, c0_ref, c1_ref, g_ref):
        dis = lax.rsqrt(c0_ref[...] + c1_ref[...] + 1.0)
        h = jnp.dot(x_ref[...], w_ref[...],
                    preferred_element_type=jnp.float32)
        g_ref[...] = h * dis

    return pl.pallas_call(
        body,
        out_shape=jax.ShapeDtypeStruct(x_pad.shape, jnp.float32),
    )(x_pad, W, c0, c1)


def _tc_finalize(p0, p1, g, c0, c1, b, gamma, beta):
    def body(p0_ref, p1_ref, g_ref, c0_ref, c1_ref, b_ref, ga_ref, be_ref,
             out_ref):
        dis = lax.rsqrt(c0_ref[...] + c1_ref[...] + 1.0)
        a = p0_ref[...] + p1_ref[...] - g_ref[...]
        r = jnp.maximum(a * dis + b_ref[...], 0.0)
        mean = jnp.mean(r, axis=0, keepdims=True)
        var = jnp.mean((r - mean) ** 2, axis=0, keepdims=True)
        out_ref[...] = (r - mean) * lax.rsqrt(var + EPS) * ga_ref[...] \
            + be_ref[...]

    return pl.pallas_call(
        body,
        out_shape=jax.ShapeDtypeStruct(p0.shape, jnp.float32),
    )(p0, p1, g, c0, c1, b, gamma, beta)


def kernel(x, edge_index, W, b, gamma, beta):
    n, d = x.shape
    e = edge_index.shape[1]
    n_pad = _pad_nodes(n)
    ktot = -(-e // (NS * CHUNK))
    k0 = (int(ktot * FAST_SHARE) + 7) // 8 * 8
    k1 = -(-(ktot - k0) // 8) * 8
    rows = NS * (k0 + k1) + abs(k0 - k1)
    e_pad = rows * CHUNK

    src = edge_index[0]
    dst = edge_index[1]
    fill = jnp.full((e_pad - e,), n, dtype=jnp.int32)
    src2 = jnp.concatenate([src, fill]).reshape(rows, CHUNK)
    dst2 = jnp.concatenate([dst, fill]).reshape(rows, CHUNK)
    ones_rows = jnp.ones((CHUNK, CHUNK), dtype=jnp.float32)
    zero_rows = jnp.zeros((CHUNK, CHUNK), dtype=jnp.float32)
    deg_parts = _make_degree_kernel(n_pad, k0, k1)(dst2, ones_rows,
                                                   zero_rows)
    c0 = deg_parts[0, :, :1]
    c1 = deg_parts[1, :, :1]

    x_pad = jnp.concatenate(
        [x, jnp.zeros((n_pad - n, d), dtype=jnp.float32)])
    g = _tc_transform(x_pad, W, c0, c1)

    parts = _make_scatter_kernel(n_pad, k0, k1, d)(g, src2, dst2)

    out = _tc_finalize(
        parts[0, :n], parts[1, :n], g[:n], c0[:n], c1[:n],
        b.reshape(1, d), gamma.reshape(1, d), beta.reshape(1, d))
    return out


# speedup vs baseline: 1.6535x; 1.6535x over previous
import functools

import jax
import jax.numpy as jnp
from jax import lax
from jax.experimental import pallas as pl
from jax.experimental.pallas import tpu as pltpu
from jax.experimental.pallas import tpu_sc as plsc

NC, NS = 2, 16
NW = NC * NS
CHUNK = 128
FAST_SHARE = 0.39
EPS = 1e-5


def _pad_nodes(n):
    per_tile = -(-n // (NS * 8)) * 8 + 8
    return per_tile * NS


def _sc_mesh():
    return plsc.VectorSubcoreMesh(core_axis_name="c", subcore_axis_name="s",
                                  num_cores=NC, num_subcores=NS)


def _make_degree_kernel(n_pad, k):
    rows_per_tile = n_pad // NS
    pieces = rows_per_tile // CHUNK

    @functools.partial(
        pl.kernel,
        out_type=jax.ShapeDtypeStruct((NC, n_pad, CHUNK), jnp.float32),
        mesh=_sc_mesh(),
        scratch_types=[
            pltpu.VMEM_SHARED((n_pad, CHUNK), jnp.float32),
            pltpu.VMEM((k, CHUNK), jnp.int32),
            pltpu.VMEM((CHUNK, CHUNK), jnp.float32),
        ],
    )
    def deg_kernel(dst_hbm, ones_hbm, zeros_hbm, out_hbm,
                   deg_sh, dst_l, buf):
        c = lax.axis_index("c")
        s = lax.axis_index("s")
        wid = c * NS + s
        base = s * rows_per_tile
        pltpu.sync_copy(zeros_hbm, buf)
        for p in range(pieces):
            pltpu.sync_copy(buf, deg_sh.at[pl.ds(base + p * CHUNK, CHUNK)])
        pltpu.sync_copy(ones_hbm, buf)
        pltpu.sync_copy(dst_hbm.at[wid], dst_l)
        plsc.subcore_barrier()

        def step(j, carry):
            pltpu.sync_copy(buf, deg_sh.at[dst_l.at[j]], add=True)
            return carry

        lax.fori_loop(0, k, step, 0)
        plsc.subcore_barrier()
        for p in range(pieces):
            pltpu.sync_copy(deg_sh.at[pl.ds(base + p * CHUNK, CHUNK)], buf)
            pltpu.sync_copy(
                buf, out_hbm.at[c].at[pl.ds(base + p * CHUNK, CHUNK)])

    return deg_kernel


def _make_scatter_kernel(n_pad, k, d):
    rows_per_tile = n_pad // NS
    pieces = rows_per_tile // CHUNK

    @functools.partial(
        pl.kernel,
        out_type=jax.ShapeDtypeStruct((NC, n_pad, d), jnp.float32),
        mesh=_sc_mesh(),
        scratch_types=[
            pltpu.VMEM_SHARED((n_pad, d), jnp.float32),
            pltpu.VMEM((k, CHUNK), jnp.int32),
            pltpu.VMEM((k, CHUNK), jnp.int32),
            pltpu.VMEM((CHUNK, d), jnp.float32),
            pltpu.SemaphoreType.DMA,
        ],
    )
    def mp_kernel(g_hbm, src_hbm, dst_hbm, out_hbm,
                  acc_sh, src_l, dst_l, buf, sem):
        c = lax.axis_index("c")
        s = lax.axis_index("s")
        wid = c * NS + s
        base = s * rows_per_tile
        pltpu.sync_copy(src_hbm.at[wid], src_l)
        pltpu.sync_copy(dst_hbm.at[wid], dst_l)
        for p in range(pieces):
            pltpu.sync_copy(g_hbm.at[pl.ds(base + p * CHUNK, CHUNK)], buf)
            pltpu.sync_copy(buf, acc_sh.at[pl.ds(base + p * CHUNK, CHUNK)])
        plsc.subcore_barrier()

        def step(j, carry):
            pltpu.async_copy(g_hbm.at[src_l.at[j]], buf, sem).wait()
            pltpu.sync_copy(buf, acc_sh.at[dst_l.at[j]], add=True)
            return carry

        lax.fori_loop(0, k, step, 0)
        plsc.subcore_barrier()
        for p in range(pieces):
            pltpu.sync_copy(acc_sh.at[pl.ds(base + p * CHUNK, CHUNK)], buf)
            pltpu.sync_copy(
                buf, out_hbm.at[c].at[pl.ds(base + p * CHUNK, CHUNK)])

    return mp_kernel


def _tc_transform(x_pad, W, c0, c1):
    def body(x_ref, w_ref, c0_ref, c1_ref, g_ref):
        dis = lax.rsqrt(c0_ref[...] + c1_ref[...] + 1.0)
        h = jnp.dot(x_ref[...], w_ref[...],
                    preferred_element_type=jnp.float32)
        g_ref[...] = h * dis

    return pl.pallas_call(
        body,
        out_shape=jax.ShapeDtypeStruct(x_pad.shape, jnp.float32),
    )(x_pad, W, c0, c1)


def _tc_finalize(p0, p1, g, c0, c1, b, gamma, beta):
    def body(p0_ref, p1_ref, g_ref, c0_ref, c1_ref, b_ref, ga_ref, be_ref,
             out_ref):
        dis = lax.rsqrt(c0_ref[...] + c1_ref[...] + 1.0)
        a = p0_ref[...] + p1_ref[...] - g_ref[...]
        r = jnp.maximum(a * dis + b_ref[...], 0.0)
        mean = jnp.mean(r, axis=0, keepdims=True)
        var = jnp.mean((r - mean) ** 2, axis=0, keepdims=True)
        out_ref[...] = (r - mean) * lax.rsqrt(var + EPS) * ga_ref[...] \
            + be_ref[...]

    return pl.pallas_call(
        body,
        out_shape=jax.ShapeDtypeStruct(p0.shape, jnp.float32),
    )(p0, p1, g, c0, c1, b, gamma, beta)


def kernel(x, edge_index, W, b, gamma, beta):
    n, d = x.shape
    e = edge_index.shape[1]
    n_pad = _pad_nodes(n)
    k = -(-e // (NW * CHUNK))
    e_pad = NW * k * CHUNK

    src = edge_index[0]
    dst = edge_index[1]
    fill = jnp.full((e_pad - e,), n, dtype=jnp.int32)
    src3 = jnp.concatenate([src, fill]).reshape(NW, k, CHUNK)
    dst3 = jnp.concatenate([dst, fill]).reshape(NW, k, CHUNK)
    ones_rows = jnp.ones((CHUNK, CHUNK), dtype=jnp.float32)
    zero_rows = jnp.zeros((CHUNK, CHUNK), dtype=jnp.float32)
    deg_parts = _make_degree_kernel(n_pad, k)(dst3, ones_rows, zero_rows)
    c0 = deg_parts[0, :, :1]
    c1 = deg_parts[1, :, :1]

    x_pad = jnp.concatenate(
        [x, jnp.zeros((n_pad - n, d), dtype=jnp.float32)])
    g = _tc_transform(x_pad, W, c0, c1)

    parts = _make_scatter_kernel(n_pad, k, d)(g, src3, dst3)

    out = _tc_finalize(
        parts[0, :n], parts[1, :n], g[:n], c0[:n], c1[:n],
        b.reshape(1, d), gamma.reshape(1, d), beta.reshape(1, d))
    return out
